# Initial kernel scaffold; baseline (speedup 1.0000x reference)
import jax
import jax.numpy as jnp
from jax.experimental import pallas as pl


def kernel(objectness, box_regression, anchors):
    raise NotImplementedError("write your pallas kernel here")


# speedup vs baseline: 16.3142x; 16.3142x over previous
import functools
import math

import jax
import jax.numpy as jnp
from jax.experimental import pallas as pl
from jax.experimental.pallas import tpu as pltpu

_PRE = 5000
_POST = 1000
_THRESH = 0.7
_CLIP = math.log(1000.0 / 16.0)
_BLK = 512
_NPAD = 5120


def _shift_right_lanes(x, s):
    return jnp.concatenate([jnp.zeros((1, s), x.dtype), x[:, :-s]], axis=1)


def _iou_cols_rows(cx1, cy1, cx2, cy2, careav, rx1, ry1, rx2, ry2, rareav):
    ltx = jnp.maximum(cx1, rx1)
    lty = jnp.maximum(cy1, ry1)
    rbx = jnp.minimum(cx2, rx2)
    rby = jnp.minimum(cy2, ry2)
    w = jnp.maximum(rbx - ltx + 1.0, 0.0)
    h = jnp.maximum(rby - lty + 1.0, 0.0)
    inter = w * h
    return inter / (careav + rareav - inter)


def _nms_body(sc_ref, br_ref, an_ref, out_ref):
    sc = sc_ref[0]
    br = br_ref[0]
    an = an_ref[0]

    ax1, ay1, ax2, ay2 = (an[0:1], an[1:2], an[2:3], an[3:4])
    w = ax2 - ax1 + 1.0
    h = ay2 - ay1 + 1.0
    ctr_x = ax1 + 0.5 * w
    ctr_y = ay1 + 0.5 * h
    dx, dy = br[0:1], br[1:2]
    dw = jnp.minimum(br[2:3], _CLIP)
    dh = jnp.minimum(br[3:4], _CLIP)
    pcx = dx * w + ctr_x
    pcy = dy * h + ctr_y
    pw = jnp.exp(dw) * w
    ph = jnp.exp(dh) * h
    px1 = pcx - 0.5 * pw
    py1 = pcy - 0.5 * ph
    px2 = pcx + 0.5 * pw - 1.0
    py2 = pcy + 0.5 * ph - 1.0
    area = (px2 - px1 + 1.0) * (py2 - py1 + 1.0)

    lane = jax.lax.broadcasted_iota(jnp.int32, (1, _NPAD), 1)
    alive = jnp.where(lane < _PRE, 1.0, 0.0)

    upper = jnp.where(
        jax.lax.broadcasted_iota(jnp.int32, (_BLK, _BLK), 0)
        < jax.lax.broadcasted_iota(jnp.int32, (_BLK, _BLK), 1), 1.0, 0.0)

    for k in range(_NPAD // _BLK):
        s0, s1 = k * _BLK, (k + 1) * _BLK
        bx1, by1, bx2, by2 = px1[:, s0:s1], py1[:, s0:s1], px2[:, s0:s1], py2[:, s0:s1]
        bar = area[:, s0:s1]
        cx1, cy1, cx2, cy2 = (jnp.transpose(bx1), jnp.transpose(by1),
                              jnp.transpose(bx2), jnp.transpose(by2))
        car = jnp.transpose(bar)
        iou = _iou_cols_rows(cx1, cy1, cx2, cy2, car, bx1, by1, bx2, by2, bar)
        m = jnp.where(iou > _THRESH, 1.0, 0.0) * upper
        blk_alive = alive[:, s0:s1]

        def fix_cond(state):
            return state[1]

        def fix_body(state):
            s, _ = state
            keep_col = jnp.transpose(blk_alive * (1.0 - s))
            s_new = jnp.max(m * keep_col, axis=0, keepdims=True)
            return s_new, jnp.any(s_new != s)

        s_fix, _ = jax.lax.while_loop(
            fix_cond, fix_body,
            (jnp.zeros((1, _BLK), jnp.float32), jnp.array(True)))
        kept_blk = blk_alive * (1.0 - s_fix)

        in_blk = (lane >= s0) & (lane < s1)
        kept_full = jnp.pad(kept_blk, ((0, 0), (s0, _NPAD - s1)))
        if s1 < _NPAD:
            rx1, ry1, rx2, ry2 = (px1[:, s1:], py1[:, s1:], px2[:, s1:], py2[:, s1:])
            rar = area[:, s1:]
            cross = _iou_cols_rows(cx1, cy1, cx2, cy2, car, rx1, ry1, rx2, ry2, rar)
            kept_col = jnp.transpose(kept_blk)
            sup = jnp.max(jnp.where(cross > _THRESH, 1.0, 0.0) * kept_col,
                          axis=0, keepdims=True)
            sup_full = jnp.pad(sup, ((0, 0), (s1, 0)))
            alive = jnp.where(in_blk, kept_full, alive * (1.0 - sup_full))
        else:
            alive = jnp.where(in_blk, kept_full, alive)

    alive_f = alive
    csum = alive_f
    s = 1
    while s < _NPAD:
        csum = csum + _shift_right_lanes(csum, s)
        s *= 2
    rank = csum - alive_f

    slot = jax.lax.broadcasted_iota(jnp.int32, (_POST, _NPAD), 0)
    onehot = jnp.where(slot == rank.astype(jnp.int32), 1.0, 0.0) * alive_f
    rows = jnp.concatenate(
        [px1, py1, px2, py2, sc, jnp.zeros((3, _NPAD), jnp.float32)], axis=0)
    out8 = jax.lax.dot_general(
        rows, onehot, (((1,), (1,)), ((), ())),
        preferred_element_type=jnp.float32)
    out_ref[0] = out8


@jax.jit
def kernel(objectness, box_regression, anchors):
    N, A, H, W = objectness.shape
    num = A * H * W
    logits = jnp.transpose(objectness, (0, 2, 3, 1)).reshape(N, num)
    scores_all = jax.nn.sigmoid(logits)
    top_scores, top_idx = jax.lax.top_k(scores_all, _PRE)

    breg = box_regression.reshape(N, A, 4, H, W)
    breg = jnp.transpose(breg, (0, 3, 4, 1, 2)).reshape(N, num, 4)
    br_g = jnp.take_along_axis(breg, top_idx[:, :, None], axis=1)
    an_g = jnp.take_along_axis(anchors, top_idx[:, :, None], axis=1)

    pad = _NPAD - _PRE
    sc_in = jnp.pad(top_scores, ((0, 0), (0, pad)))[:, None, :]
    br_in = jnp.pad(jnp.transpose(br_g, (0, 2, 1)), ((0, 0), (0, 0), (0, pad)))
    an_in = jnp.pad(jnp.transpose(an_g, (0, 2, 1)), ((0, 0), (0, 0), (0, pad)))

    out8 = pl.pallas_call(
        _nms_body,
        grid=(N,),
        in_specs=[
            pl.BlockSpec((1, 1, _NPAD), lambda i: (i, 0, 0)),
            pl.BlockSpec((1, 4, _NPAD), lambda i: (i, 0, 0)),
            pl.BlockSpec((1, 4, _NPAD), lambda i: (i, 0, 0)),
        ],
        out_specs=pl.BlockSpec((1, 8, _POST), lambda i: (i, 0, 0)),
        out_shape=jax.ShapeDtypeStruct((N, 8, _POST), jnp.float32),
    )(sc_in, br_in, an_in)

    return jnp.transpose(out8, (0, 2, 1))[..., :5]
